# baseline (speedup 1.0000x reference)
import functools

import jax
import jax.numpy as jnp
from jax import lax
from jax.experimental import pallas as pl
from jax.experimental.pallas import tpu as pltpu
from jax.experimental.pallas import tpu_sc as plsc

T_LEN = 50
B_LEN = 4096
D = 16
NSTATES = 1000000
NC, NS = 2, 16
NW = NC * NS

_mesh = plsc.VectorSubcoreMesh(core_axis_name="c", subcore_axis_name="s")


def _do_step(r, xt_hbm, mt_hbm, out_hbm, xv, outv, gsems, osems):
    pltpu.sync_copy(xt_hbm.at[r], xv)

    g = [None, None]
    w = [None, None]
    for j in range(D):
        jb = j % 2
        if w[jb] is not None:
            w[jb].wait()
        g[jb] = pltpu.async_copy(
            mt_hbm.at[j].at[plsc.Indices(xv)],
            outv.at[jb],
            gsems[jb],
        )
        if j > 0:
            pb = jb ^ 1
            g[pb].wait()
            w[pb] = pltpu.async_copy(outv.at[pb], out_hbm.at[r, j - 1], osems[pb])
    lb = (D - 1) % 2
    g[lb].wait()
    w[lb] = pltpu.async_copy(outv.at[lb], out_hbm.at[r, D - 1], osems[lb])
    w[lb ^ 1].wait()
    w[lb].wait()


def _gather_body(xt_hbm, mt_hbm, out_hbm, xv, outv, gsem0, gsem1, osem0, osem1):
    c = lax.axis_index("c")
    s = lax.axis_index("s")
    wid = s * NC + c
    gsems = [gsem0, gsem1]
    osems = [osem0, osem1]

    _do_step(wid, xt_hbm, mt_hbm, out_hbm, xv, outv, gsems, osems)

    @pl.when(wid + NW < T_LEN)
    def _():
        _do_step(wid + NW, xt_hbm, mt_hbm, out_hbm, xv, outv, gsems, osems)


_sc_gather = functools.partial(
    pl.kernel,
    mesh=_mesh,
    out_type=jax.ShapeDtypeStruct((T_LEN, D, B_LEN), jnp.float32),
    scratch_types=[
        pltpu.VMEM((B_LEN,), jnp.int32),
        pltpu.VMEM((2, B_LEN), jnp.float32),
        pltpu.SemaphoreType.DMA,
        pltpu.SemaphoreType.DMA,
        pltpu.SemaphoreType.DMA,
        pltpu.SemaphoreType.DMA,
    ],
    compiler_params=pltpu.CompilerParams(use_tc_tiling_on_sc=False),
)(_gather_body)


def _scale_body(sig_ref, out_ref):
    sig = sig_ref[0, :]
    out_ref[...] = jnp.broadcast_to(sig[None, :, None], out_ref.shape)


def _scale_bcast(sigma):
    return pl.pallas_call(
        _scale_body,
        out_shape=jax.ShapeDtypeStruct((T_LEN, D, B_LEN), jnp.float32),
        grid=(T_LEN // 5,),
        in_specs=[pl.BlockSpec((1, D), lambda i: (0, 0))],
        out_specs=pl.BlockSpec((5, D, B_LEN), lambda i: (i, 0, 0)),
    )(sigma.reshape(1, D))


def kernel(x, u, t, means, sigma):
    xt = jnp.swapaxes(x.astype(jnp.int32), 0, 1)
    mt = jnp.swapaxes(means, 0, 1)
    outp = _sc_gather(xt, mt)
    loc = jnp.transpose(outp, (2, 0, 1))
    scale = jnp.transpose(_scale_bcast(sigma), (2, 0, 1))
    return (loc, scale)


# speedup vs baseline: 1.0395x; 1.0395x over previous
import functools

import jax
import jax.numpy as jnp
from jax import lax
from jax.experimental import pallas as pl
from jax.experimental.pallas import tpu as pltpu
from jax.experimental.pallas import tpu_sc as plsc

B_LEN = 4096
T_LEN = 50
D = 16
N = B_LEN * T_LEN
EPB = 8
W128 = EPB * D
NBLK = 1000000 // EPB
NC, NS = 2, 16
NW = NC * NS
PER_W = N // NW
CHUNK = 400
NCHUNK = PER_W // CHUNK

_mesh = plsc.VectorSubcoreMesh(core_axis_name="c", subcore_axis_name="s")


def _gather_body(tab_hbm, idx_hbm, out_hbm, idx_v, rows0, rows1,
                 gsem, osem0, osem1):
    wid = lax.axis_index("s") * NC + lax.axis_index("c")
    base = wid * PER_W
    pltpu.sync_copy(idx_hbm.at[pl.ds(base, PER_W)], idx_v)

    rows = [rows0, rows1]
    osem = [osem0, osem1]
    w = [None, None]
    for k in range(NCHUNK):
        b = k % 2
        if w[b] is not None:
            w[b].wait()
        pltpu.async_copy(
            tab_hbm.at[idx_v.at[pl.ds(k * CHUNK, CHUNK)]], rows[b], gsem
        ).wait()
        w[b] = pltpu.async_copy(
            rows[b], out_hbm.at[pl.ds(base + k * CHUNK, CHUNK)], osem[b]
        )
    w[0].wait()
    w[1].wait()


_sc_gather = functools.partial(
    pl.kernel,
    mesh=_mesh,
    out_type=jax.ShapeDtypeStruct((N, W128), jnp.float32),
    scratch_types=[
        pltpu.VMEM((PER_W,), jnp.int32),
        pltpu.VMEM((CHUNK, W128), jnp.float32),
        pltpu.VMEM((CHUNK, W128), jnp.float32),
        pltpu.SemaphoreType.DMA,
        pltpu.SemaphoreType.DMA,
        pltpu.SemaphoreType.DMA,
    ],
)(_gather_body)


BLK = 2048


def _extract_body(g_ref, sel_ref, sig_ref, loc_ref, scale_ref):
    g3 = g_ref[...].reshape(BLK, EPB, D)
    s = sel_ref[...]
    m = (lax.broadcasted_iota(jnp.int32, (BLK, EPB), 1) == s[:, None])
    loc_ref[...] = jnp.sum(g3 * m.astype(jnp.float32)[:, :, None], axis=1)
    scale_ref[...] = jnp.broadcast_to(sig_ref[0, :], (BLK, D))


def _tc_extract(blocks, sel, sigma):
    return pl.pallas_call(
        _extract_body,
        out_shape=(
            jax.ShapeDtypeStruct((N, D), jnp.float32),
            jax.ShapeDtypeStruct((N, D), jnp.float32),
        ),
        grid=(N // BLK,),
        in_specs=[
            pl.BlockSpec((BLK, W128), lambda i: (i, 0)),
            pl.BlockSpec((BLK,), lambda i: (i,)),
            pl.BlockSpec((1, D), lambda i: (0, 0)),
        ],
        out_specs=(
            pl.BlockSpec((BLK, D), lambda i: (i, 0)),
            pl.BlockSpec((BLK, D), lambda i: (i, 0)),
        ),
    )(blocks, sel, sigma.reshape(1, D))


def kernel(x, u, t, means, sigma):
    xi = x.astype(jnp.int32).reshape(N)
    bidx = xi >> 3
    sel = xi & 7
    tab = means.reshape(NBLK, W128)
    blocks = _sc_gather(tab, bidx)
    loc, scale = _tc_extract(blocks, sel, sigma)
    return (loc.reshape(B_LEN, T_LEN, D), scale.reshape(B_LEN, T_LEN, D))


# speedup vs baseline: 1.1326x; 1.0895x over previous
import functools

import jax
import jax.numpy as jnp
from jax import lax
from jax.experimental import pallas as pl
from jax.experimental.pallas import tpu as pltpu
from jax.experimental.pallas import tpu_sc as plsc

B_LEN = 4096
T_LEN = 50
D = 16
N = B_LEN * T_LEN
EPB = 8
W128 = EPB * D
NBLK = 1000000 // EPB
NC, NS = 2, 16
NW = NC * NS
PER_W = N // NW
CHUNK = 400
NCHUNK = PER_W // CHUNK

_mesh = plsc.VectorSubcoreMesh(core_axis_name="c", subcore_axis_name="s")


def _gather_body(tab_hbm, idx_hbm, out_hbm, idx_v, rows0, rows1,
                 gsem, osem0, osem1):
    wid = lax.axis_index("s") * NC + lax.axis_index("c")
    base = wid * PER_W
    pltpu.sync_copy(idx_hbm.at[pl.ds(base, PER_W)], idx_v)

    rows = [rows0, rows1]
    osem = [osem0, osem1]
    w = [None, None]
    for k in range(NCHUNK):
        b = k % 2
        if w[b] is not None:
            w[b].wait()
        pltpu.async_copy(
            tab_hbm.at[idx_v.at[pl.ds(k * CHUNK, CHUNK)]], rows[b], gsem
        ).wait()
        w[b] = pltpu.async_copy(
            rows[b], out_hbm.at[pl.ds(base + k * CHUNK, CHUNK)], osem[b]
        )
    w[0].wait()
    w[1].wait()


_sc_gather = functools.partial(
    pl.kernel,
    mesh=_mesh,
    out_type=jax.ShapeDtypeStruct((N, W128), jnp.float32),
    scratch_types=[
        pltpu.VMEM((PER_W,), jnp.int32),
        pltpu.VMEM((CHUNK, W128), jnp.float32),
        pltpu.VMEM((CHUNK, W128), jnp.float32),
        pltpu.SemaphoreType.DMA,
        pltpu.SemaphoreType.DMA,
        pltpu.SemaphoreType.DMA,
    ],
)(_gather_body)


BLK = 2048


def _extract_body(g_ref, sel_ref, sig_ref, loc_ref, scale_ref):
    g3 = g_ref[...].reshape(BLK, EPB, D)
    s = sel_ref[...]
    m = (lax.broadcasted_iota(jnp.int32, (BLK, EPB), 1) == s[:, None])
    loc = jnp.sum(g3 * m.astype(jnp.float32)[:, :, None], axis=1)
    loc_ref[...] = loc.T
    scale_ref[...] = jnp.broadcast_to(sig_ref[0, :][:, None], (D, BLK))


def _tc_extract(blocks, sel, sigma):
    return pl.pallas_call(
        _extract_body,
        out_shape=(
            jax.ShapeDtypeStruct((D, N), jnp.float32),
            jax.ShapeDtypeStruct((D, N), jnp.float32),
        ),
        grid=(N // BLK,),
        in_specs=[
            pl.BlockSpec((BLK, W128), lambda i: (i, 0)),
            pl.BlockSpec((BLK,), lambda i: (i,)),
            pl.BlockSpec((1, D), lambda i: (0, 0)),
        ],
        out_specs=(
            pl.BlockSpec((D, BLK), lambda i: (0, i)),
            pl.BlockSpec((D, BLK), lambda i: (0, i)),
        ),
    )(blocks, sel, sigma.reshape(1, D))


def kernel(x, u, t, means, sigma):
    xi = x.astype(jnp.int32).reshape(N)
    bidx = xi >> 3
    sel = xi & 7
    tab = means.reshape(NBLK, W128)
    blocks = _sc_gather(tab, bidx)
    loc_t, scale_t = _tc_extract(blocks, sel, sigma)
    loc = loc_t.T.reshape(B_LEN, T_LEN, D)
    scale = scale_t.T.reshape(B_LEN, T_LEN, D)
    return (loc, scale)


# speedup vs baseline: 1.1331x; 1.0005x over previous
import functools

import jax
import jax.numpy as jnp
from jax import lax
from jax.experimental import pallas as pl
from jax.experimental.pallas import tpu as pltpu
from jax.experimental.pallas import tpu_sc as plsc

B_LEN = 4096
T_LEN = 50
D = 16
N = B_LEN * T_LEN
EPB = 8
W128 = EPB * D
NBLK = 1000000 // EPB
NC, NS = 2, 16
NW = NC * NS
PER_W = N // NW
CHUNK = 400
NCHUNK = PER_W // CHUNK

_mesh = plsc.VectorSubcoreMesh(core_axis_name="c", subcore_axis_name="s")


def _gather_body(tab_hbm, idx_hbm, out_hbm, idx_v, rows0, rows1,
                 gsem, osem0, osem1):
    wid = lax.axis_index("s") * NC + lax.axis_index("c")
    base = wid * PER_W
    pltpu.sync_copy(idx_hbm.at[pl.ds(base, PER_W)], idx_v)

    rows = [rows0, rows1]
    osem = [osem0, osem1]
    w = [None, None]
    for k in range(NCHUNK):
        b = k % 2
        if w[b] is not None:
            w[b].wait()
        pltpu.async_copy(
            tab_hbm.at[idx_v.at[pl.ds(k * CHUNK, CHUNK)]], rows[b], gsem
        ).wait()
        w[b] = pltpu.async_copy(
            rows[b], out_hbm.at[pl.ds(base + k * CHUNK, CHUNK)], osem[b]
        )
    w[0].wait()
    w[1].wait()


_sc_gather = functools.partial(
    pl.kernel,
    mesh=_mesh,
    out_type=jax.ShapeDtypeStruct((N, W128), jnp.float32),
    scratch_types=[
        pltpu.VMEM((PER_W,), jnp.int32),
        pltpu.VMEM((CHUNK, W128), jnp.float32),
        pltpu.VMEM((CHUNK, W128), jnp.float32),
        pltpu.SemaphoreType.DMA,
        pltpu.SemaphoreType.DMA,
        pltpu.SemaphoreType.DMA,
    ],
    compiler_params=pltpu.CompilerParams(use_tc_tiling_on_sc=True),
)(_gather_body)


BLK = 2048


def _extract_body(g_ref, sel_ref, sig_ref, loc_ref, scale_ref):
    g3 = g_ref[...].reshape(BLK, EPB, D)
    s = sel_ref[...]
    m = (lax.broadcasted_iota(jnp.int32, (BLK, EPB), 1) == s[:, None])
    loc = jnp.sum(g3 * m.astype(jnp.float32)[:, :, None], axis=1)
    loc_ref[...] = loc.T
    scale_ref[...] = jnp.broadcast_to(sig_ref[0, :][:, None], (D, BLK))


def _tc_extract(blocks, sel, sigma):
    return pl.pallas_call(
        _extract_body,
        out_shape=(
            jax.ShapeDtypeStruct((D, N), jnp.float32),
            jax.ShapeDtypeStruct((D, N), jnp.float32),
        ),
        grid=(N // BLK,),
        in_specs=[
            pl.BlockSpec((BLK, W128), lambda i: (i, 0)),
            pl.BlockSpec((BLK,), lambda i: (i,)),
            pl.BlockSpec((1, D), lambda i: (0, 0)),
        ],
        out_specs=(
            pl.BlockSpec((D, BLK), lambda i: (0, i)),
            pl.BlockSpec((D, BLK), lambda i: (0, i)),
        ),
    )(blocks, sel, sigma.reshape(1, D))


def kernel(x, u, t, means, sigma):
    xi = x.astype(jnp.int32).reshape(N)
    bidx = xi >> 3
    sel = xi & 7
    tab = means.reshape(NBLK, W128)
    blocks = _sc_gather(tab, bidx)
    loc_t, scale_t = _tc_extract(blocks, sel, sigma)
    loc = loc_t.T.reshape(B_LEN, T_LEN, D)
    scale = scale_t.T.reshape(B_LEN, T_LEN, D)
    return (loc, scale)
